# baseline (speedup 1.0000x reference)
import jax
import jax.numpy as jnp
from jax.experimental import pallas as pl
from jax.experimental.pallas import tpu as pltpu


_BLOCK_ROWS = 2048


def _fill(out_hbm, ztile, sems):
    n_chunks = out_hbm.shape[0] // ztile.shape[0]
    ztile[...] = jnp.zeros_like(ztile)
    copies = [
        pltpu.make_async_copy(
            ztile,
            out_hbm.at[pl.ds(i * ztile.shape[0], ztile.shape[0]), :],
            sems.at[i],
        )
        for i in range(n_chunks)
    ]
    for c in copies:
        c.start()
    for c in copies:
        c.wait()


def kernel(syndrome):
    rows, cols = syndrome.shape
    block_rows = _BLOCK_ROWS if rows % _BLOCK_ROWS == 0 else rows
    n_chunks = rows // block_rows
    return pl.pallas_call(
        _fill,
        out_specs=pl.BlockSpec(memory_space=pl.ANY),
        out_shape=jax.ShapeDtypeStruct((rows, cols), syndrome.dtype),
        scratch_shapes=[
            pltpu.VMEM((block_rows, cols), syndrome.dtype),
            pltpu.SemaphoreType.DMA((n_chunks,)),
        ],
    )()


# speedup vs baseline: 1.0055x; 1.0055x over previous
import jax
import jax.numpy as jnp
from jax.experimental import pallas as pl
from jax.experimental.pallas import tpu as pltpu


_BLOCK_ROWS = 512


def _fill(out_hbm, ztile, sems):
    n_chunks = out_hbm.shape[0] // ztile.shape[0]
    ztile[...] = jnp.zeros_like(ztile)
    copies = [
        pltpu.make_async_copy(
            ztile,
            out_hbm.at[pl.ds(i * ztile.shape[0], ztile.shape[0]), :],
            sems.at[i],
        )
        for i in range(n_chunks)
    ]
    for c in copies:
        c.start()
    for c in copies:
        c.wait()


def kernel(syndrome):
    rows, cols = syndrome.shape
    block_rows = _BLOCK_ROWS if rows % _BLOCK_ROWS == 0 else rows
    n_chunks = rows // block_rows
    return pl.pallas_call(
        _fill,
        out_specs=pl.BlockSpec(memory_space=pl.ANY),
        out_shape=jax.ShapeDtypeStruct((rows, cols), syndrome.dtype),
        scratch_shapes=[
            pltpu.VMEM((block_rows, cols), syndrome.dtype),
            pltpu.SemaphoreType.DMA((n_chunks,)),
        ],
    )()


# speedup vs baseline: 1.0556x; 1.0498x over previous
import jax
import jax.numpy as jnp
from jax.experimental import pallas as pl
from jax.experimental.pallas import tpu as pltpu


_BLOCK_ROWS = 2048


def _zero_fill(out_ref):
    out_ref[...] = jnp.zeros_like(out_ref)


def kernel(syndrome):
    rows, cols = syndrome.shape
    block_rows = _BLOCK_ROWS if rows % _BLOCK_ROWS == 0 else rows
    return pl.pallas_call(
        _zero_fill,
        grid=(rows // block_rows,),
        out_specs=pl.BlockSpec((block_rows, cols), lambda i: (i, 0)),
        out_shape=jax.ShapeDtypeStruct((rows, cols), syndrome.dtype),
        compiler_params=pltpu.CompilerParams(
            dimension_semantics=("parallel",),
        ),
    )()
